# baseline (speedup 1.0000x reference)
import jax
import jax.numpy as jnp
from jax.experimental import pallas as pl
from jax.experimental.pallas import tpu as pltpu


def _gcn_kernel(x_ref, w_ref, b_ref, adj_ref, out_ref, s_ref):
    @pl.when(pl.program_id(0) == 0)
    def _():
        s_ref[...] = jnp.dot(x_ref[...], w_ref[...],
                             preferred_element_type=jnp.float32)

    acc = jnp.dot(adj_ref[...], s_ref[...],
                  preferred_element_type=jnp.float32)
    out_ref[...] = jnp.maximum(acc + b_ref[...], 0.0)


def kernel(x, adj, W, b):
    N, F = x.shape
    H = W.shape[1]

    BR = 400
    out = pl.pallas_call(
        _gcn_kernel,
        grid=(N // BR,),
        in_specs=[
            pl.BlockSpec((N, F), lambda i: (0, 0)),
            pl.BlockSpec((F, H), lambda i: (0, 0)),
            pl.BlockSpec((1, H), lambda i: (0, 0)),
            pl.BlockSpec((BR, N), lambda i: (i, 0)),
        ],
        out_specs=pl.BlockSpec((BR, H), lambda i: (i, 0)),
        out_shape=jax.ShapeDtypeStruct((N, H), jnp.float32),
        scratch_shapes=[pltpu.VMEM((N, H), jnp.float32)],
        compiler_params=pltpu.CompilerParams(
            dimension_semantics=("arbitrary",),
            vmem_limit_bytes=100 * 1024 * 1024,
        ),
    )(x, W, b.reshape(1, H), adj)
    return out


# speedup vs baseline: 1.0024x; 1.0024x over previous
import jax
import jax.numpy as jnp
from jax.experimental import pallas as pl
from jax.experimental.pallas import tpu as pltpu


def _gcn_body(nblk, br, x_ref, w_ref, b_ref, adj_hbm, out_ref,
              s_ref, buf_ref, sems):
    nbuf = buf_ref.shape[0]

    def start_copy(i, slot):
        pltpu.make_async_copy(
            adj_hbm.at[pl.ds(i * br, br), :],
            buf_ref.at[slot],
            sems.at[slot],
        ).start()

    for k in range(nbuf):
        start_copy(k, k)

    s_ref[...] = jnp.dot(x_ref[...], w_ref[...],
                         preferred_element_type=jnp.float32)

    def loop(i, carry):
        slot = jax.lax.rem(i, nbuf)
        pltpu.make_async_copy(
            adj_hbm.at[pl.ds(i * br, br), :],
            buf_ref.at[slot],
            sems.at[slot],
        ).wait()
        acc = jnp.dot(buf_ref[slot], s_ref[...],
                      preferred_element_type=jnp.float32)
        out_ref[pl.ds(i * br, br), :] = jnp.maximum(acc + b_ref[...], 0.0)

        @pl.when(i + nbuf < nblk)
        def _():
            start_copy(i + nbuf, slot)

        return carry

    jax.lax.fori_loop(0, nblk, loop, 0)


def kernel(x, adj, W, b):
    N, F = x.shape
    H = W.shape[1]

    BR = 200
    NBUF = 4
    nblk = N // BR

    import functools
    out = pl.pallas_call(
        functools.partial(_gcn_body, nblk, BR),
        in_specs=[
            pl.BlockSpec(memory_space=pltpu.VMEM),
            pl.BlockSpec(memory_space=pltpu.VMEM),
            pl.BlockSpec(memory_space=pltpu.VMEM),
            pl.BlockSpec(memory_space=pltpu.HBM),
        ],
        out_specs=pl.BlockSpec(memory_space=pltpu.VMEM),
        out_shape=jax.ShapeDtypeStruct((N, H), jnp.float32),
        scratch_shapes=[
            pltpu.VMEM((N, H), jnp.float32),
            pltpu.VMEM((NBUF, BR, N), jnp.float32),
            pltpu.SemaphoreType.DMA((NBUF,)),
        ],
    )(x, W, b.reshape(1, H), adj)
    return out
